# baseline (speedup 1.0000x reference)
import jax
import jax.numpy as jnp
from jax.experimental import pallas as pl
from jax.experimental.pallas import tpu as pltpu

_NUM_EMB = 1024
_DIM = 16384
_BATCH = 8
_BK = 64
_NW = 4
_CW = _DIM // _NW


def _vq_body(x4_ref, *refs):
    w_refs = refs[:_NW]
    w_any = refs[_NW]
    (idx_ref, loss_ref, q4_ref,
     flat_ref, q2_ref, minval_ref, minidx_ref,
     idx_smem, sem_idx, sem_rows) = refs[_NW + 1:]
    k = pl.program_id(0)
    nk = pl.num_programs(0)

    @pl.when(k == 0)
    def _():
        flat_ref[...] = x4_ref[...].reshape(_BATCH, _DIM)

    flat = flat_ref[...]
    dot = None
    w_sq = None
    for j in range(_NW):
        wj = w_refs[j][...]
        dj = jax.lax.dot_general(
            flat[:, j * _CW:(j + 1) * _CW], wj, (((1,), (1,)), ((), ())),
            preferred_element_type=jnp.float32)
        sj = jnp.sum(wj * wj, axis=1)
        dot = dj if dot is None else dot + dj
        w_sq = sj if w_sq is None else w_sq + sj
    d2p = w_sq[None, :] - 2.0 * dot
    local_min = jnp.min(d2p, axis=1, keepdims=True)
    lane = jax.lax.broadcasted_iota(jnp.int32, d2p.shape, 1)
    local_arg = jnp.min(
        jnp.where(d2p == local_min, lane, _NUM_EMB), axis=1, keepdims=True
    ) + k * _BK

    @pl.when(k == 0)
    def _():
        minval_ref[...] = local_min
        minidx_ref[...] = local_arg

    @pl.when(k > 0)
    def _():
        better = local_min < minval_ref[...]
        minval_ref[...] = jnp.where(better, local_min, minval_ref[...])
        minidx_ref[...] = jnp.where(better, local_arg, minidx_ref[...])

    @pl.when(k == nk - 1)
    def _():
        x_sq = jnp.sum(flat * flat, axis=1, keepdims=True)
        d2min = minval_ref[...] + x_sq
        loss_ref[...] = (1.25 / (_BATCH * _DIM)) * jnp.sum(
            d2min, keepdims=True)
        idx_ref[...] = minidx_ref[...]
        pltpu.async_copy(minidx_ref, idx_smem, sem_idx).wait()
        copies = [
            pltpu.async_copy(
                w_any.at[idx_smem[b, 0]], q2_ref.at[b], sem_rows.at[b])
            for b in range(_BATCH)
        ]
        for c in copies:
            c.wait()
        q4_ref[...] = q2_ref[...].reshape(_BATCH, _DIM // 64, 8, 8)


def _vq_pallas(inputs, emb_weight):
    grid = _NUM_EMB // _BK
    idx, loss, quantized = pl.pallas_call(
        _vq_body,
        grid=(grid,),
        in_specs=[
            pl.BlockSpec(inputs.shape, lambda k: (0, 0, 0, 0)),
        ] + [
            pl.BlockSpec((_BK, _CW), lambda k, j=j: (k, j))
            for j in range(_NW)
        ] + [
            pl.BlockSpec(memory_space=pltpu.MemorySpace.HBM),
        ],
        out_specs=[
            pl.BlockSpec((_BATCH, 1), lambda k: (0, 0)),
            pl.BlockSpec((1, 1), lambda k: (0, 0)),
            pl.BlockSpec(inputs.shape, lambda k: (0, 0, 0, 0)),
        ],
        out_shape=[
            jax.ShapeDtypeStruct((_BATCH, 1), jnp.int32),
            jax.ShapeDtypeStruct((1, 1), jnp.float32),
            jax.ShapeDtypeStruct(inputs.shape, jnp.float32),
        ],
        scratch_shapes=[
            pltpu.VMEM((_BATCH, _DIM), jnp.float32),
            pltpu.VMEM((_BATCH, _DIM), jnp.float32),
            pltpu.VMEM((_BATCH, 1), jnp.float32),
            pltpu.VMEM((_BATCH, 1), jnp.int32),
            pltpu.SMEM((_BATCH, 1), jnp.int32),
            pltpu.SemaphoreType.DMA,
            pltpu.SemaphoreType.DMA((_BATCH,)),
        ],
    )(inputs, *([emb_weight] * _NW), emb_weight)
    return idx, loss, quantized


def kernel(inputs, emb_weight):
    idx, loss, quantized = _vq_pallas(inputs, emb_weight)
    return (
        quantized,
        loss.reshape(()),
        idx,
    )


# speedup vs baseline: 1.2969x; 1.2969x over previous
import jax
import jax.numpy as jnp
from jax.experimental import pallas as pl
from jax.experimental.pallas import tpu as pltpu

_NUM_EMB = 1024
_DIM = 16384
_BATCH = 8
_BK = 128
_NW = 4
_CW = _DIM // _NW


def _vq_body(x4_ref, *refs):
    w_refs = refs[:_NW]
    w_any = refs[_NW]
    (idx_ref, loss_ref, q4_ref,
     flat_ref, q2_ref, minval_ref, minidx_ref,
     idx_smem, sem_idx, sem_rows) = refs[_NW + 1:]
    k = pl.program_id(0)
    nk = pl.num_programs(0)

    @pl.when(k == 0)
    def _():
        flat_ref[...] = jnp.transpose(
            x4_ref[...], (0, 3, 1, 2)).reshape(_BATCH, _DIM)

    flat = flat_ref[...]
    dot = None
    w_sq = None
    for j in range(_NW):
        wj = w_refs[j][...]
        dj = jax.lax.dot_general(
            flat[:, j * _CW:(j + 1) * _CW], wj, (((1,), (1,)), ((), ())),
            preferred_element_type=jnp.float32)
        sj = jnp.sum(wj * wj, axis=1)
        dot = dj if dot is None else dot + dj
        w_sq = sj if w_sq is None else w_sq + sj
    d2p = w_sq[None, :] - 2.0 * dot
    local_min = jnp.min(d2p, axis=1, keepdims=True)
    lane = jax.lax.broadcasted_iota(jnp.int32, d2p.shape, 1)
    local_arg = jnp.min(
        jnp.where(d2p == local_min, lane, _NUM_EMB), axis=1, keepdims=True
    ) + k * _BK

    @pl.when(k == 0)
    def _():
        minval_ref[...] = local_min
        minidx_ref[...] = local_arg

    @pl.when(k > 0)
    def _():
        better = local_min < minval_ref[...]
        minval_ref[...] = jnp.where(better, local_min, minval_ref[...])
        minidx_ref[...] = jnp.where(better, local_arg, minidx_ref[...])

    @pl.when(k == nk - 1)
    def _():
        x_sq = jnp.sum(flat * flat, axis=1, keepdims=True)
        d2min = minval_ref[...] + x_sq
        loss_ref[...] = (1.25 / (_BATCH * _DIM)) * jnp.sum(
            d2min, keepdims=True)
        idx_ref[...] = minidx_ref[...]
        pltpu.async_copy(minidx_ref, idx_smem, sem_idx).wait()
        copies = [
            pltpu.async_copy(
                w_any.at[idx_smem[b, 0]], q2_ref.at[b], sem_rows.at[b])
            for b in range(_BATCH)
        ]
        for c in copies:
            c.wait()
        q4_ref[...] = jnp.transpose(
            q2_ref[...].reshape(_BATCH, _DIM // 64, 8, 8), (0, 2, 3, 1))


def _vq_pallas(inputs, emb_weight):
    x_t = jnp.transpose(inputs, (0, 2, 3, 1))
    grid = _NUM_EMB // _BK
    idx, loss, quantized = pl.pallas_call(
        _vq_body,
        grid=(grid,),
        in_specs=[
            pl.BlockSpec(x_t.shape, lambda k: (0, 0, 0, 0)),
        ] + [
            pl.BlockSpec((_BK, _CW), lambda k, j=j: (k, j))
            for j in range(_NW)
        ] + [
            pl.BlockSpec(memory_space=pltpu.MemorySpace.HBM),
        ],
        out_specs=[
            pl.BlockSpec((_BATCH, 1), lambda k: (0, 0)),
            pl.BlockSpec((1, 1), lambda k: (0, 0)),
            pl.BlockSpec(x_t.shape, lambda k: (0, 0, 0, 0)),
        ],
        out_shape=[
            jax.ShapeDtypeStruct((_BATCH, 1), jnp.int32),
            jax.ShapeDtypeStruct((1, 1), jnp.float32),
            jax.ShapeDtypeStruct(x_t.shape, jnp.float32),
        ],
        scratch_shapes=[
            pltpu.VMEM((_BATCH, _DIM), jnp.float32),
            pltpu.VMEM((_BATCH, _DIM), jnp.float32),
            pltpu.VMEM((_BATCH, 1), jnp.float32),
            pltpu.VMEM((_BATCH, 1), jnp.int32),
            pltpu.SMEM((_BATCH, 1), jnp.int32),
            pltpu.SemaphoreType.DMA,
            pltpu.SemaphoreType.DMA((_BATCH,)),
        ],
    )(x_t, *([emb_weight] * _NW), emb_weight)
    return idx, loss, jnp.transpose(quantized, (0, 3, 1, 2))


def kernel(inputs, emb_weight):
    idx, loss, quantized = _vq_pallas(inputs, emb_weight)
    return (
        quantized,
        loss.reshape(()),
        idx,
    )
